# Initial kernel scaffold; baseline (speedup 1.0000x reference)
import functools

import jax
import jax.numpy as jnp
from jax import lax
from jax.experimental import pallas as pl
from jax.experimental.pallas import tpu as pltpu
from jax.experimental.pallas import tpu_sc as plsc


def _tc_pre_body(x_ref, w_ref, asrc_ref, adst_ref, h_ref, s_ref, d_ref):
    h = jnp.dot(x_ref[...], w_ref[...], preferred_element_type=jnp.float32)
    h_ref[...] = h
    s_ref[...] = jnp.dot(h, asrc_ref[...], preferred_element_type=jnp.float32)
    d_ref[...] = jnp.dot(h, adst_ref[...], preferred_element_type=jnp.float32)


def _tc_pre(x, W, Asrc, Adst):
    N, F = x.shape
    BLK = 1000
    return pl.pallas_call(
        _tc_pre_body,
        grid=(N // BLK,),
        in_specs=[
            pl.BlockSpec((BLK, F), lambda i: (i, 0)),
            pl.BlockSpec((F, 128), lambda i: (0, 0)),
            pl.BlockSpec((128, 16), lambda i: (0, 0)),
            pl.BlockSpec((128, 16), lambda i: (0, 0)),
        ],
        out_specs=[
            pl.BlockSpec((BLK, 128), lambda i: (i, 0)),
            pl.BlockSpec((BLK, 16), lambda i: (i, 0)),
            pl.BlockSpec((BLK, 16), lambda i: (i, 0)),
        ],
        out_shape=[
            jax.ShapeDtypeStruct((N, 128), jnp.float32),
            jax.ShapeDtypeStruct((N, 16), jnp.float32),
            jax.ShapeDtypeStruct((N, 16), jnp.float32),
        ],
    )(x, W, Asrc, Adst)


def _tc_mid_body(acc_ref, r_ref, b_ref, g_ref, be_ref, w_ref, asrc_ref,
                 adst_ref, h2_ref, s_ref, d_ref):
    P = acc_ref[0] + acc_ref[1]
    hagg = P[:, :128]
    denb = jnp.dot(P[:, 128:144], r_ref[...], preferred_element_type=jnp.float32)
    y = jnp.where(denb > 0.0, hagg / denb, 0.0) + b_ref[...]
    mu = jnp.mean(y, axis=0, keepdims=True)
    var = jnp.mean((y - mu) ** 2, axis=0, keepdims=True)
    yn = (y - mu) * lax.rsqrt(var + 1e-5) * g_ref[...] + be_ref[...]
    yr = jnp.maximum(yn, 0.0)
    h2 = jnp.dot(yr, w_ref[...], preferred_element_type=jnp.float32)
    h2_ref[...] = h2
    s_ref[...] = jnp.dot(h2, asrc_ref[...], preferred_element_type=jnp.float32)
    d_ref[...] = jnp.dot(h2, adst_ref[...], preferred_element_type=jnp.float32)


def _tc_mid(acc, R, bias, gamma, beta, W, Asrc, Adst):
    N = acc.shape[1]
    return pl.pallas_call(
        _tc_mid_body,
        out_shape=[
            jax.ShapeDtypeStruct((N, 128), jnp.float32),
            jax.ShapeDtypeStruct((N, 16), jnp.float32),
            jax.ShapeDtypeStruct((N, 16), jnp.float32),
        ],
    )(acc, R, bias.reshape(1, 128), gamma.reshape(1, 128),
      beta.reshape(1, 128), W, Asrc, Adst)


def _tc_post_body(acc_ref, r_ref, b_ref, g_ref, be_ref, fcw_ref, fcb_ref,
                  out_ref):
    P = acc_ref[0] + acc_ref[1]
    hagg = P[:, :128]
    denb = jnp.dot(P[:, 128:144], r_ref[...], preferred_element_type=jnp.float32)
    y = jnp.where(denb > 0.0, hagg / denb, 0.0) + b_ref[...]
    mu = jnp.mean(y, axis=0, keepdims=True)
    var = jnp.mean((y - mu) ** 2, axis=0, keepdims=True)
    yn = (y - mu) * lax.rsqrt(var + 1e-5) * g_ref[...] + be_ref[...]
    yr = jnp.maximum(yn, 0.0)
    o = jnp.dot(yr, fcw_ref[...], preferred_element_type=jnp.float32)
    out_ref[...] = jax.nn.sigmoid(o + fcb_ref[0, 0])


def _tc_post(acc, R, bias, gamma, beta, fcW, fcb):
    N = acc.shape[1]
    return pl.pallas_call(
        _tc_post_body,
        out_shape=jax.ShapeDtypeStruct((N, 1), jnp.float32),
    )(acc, R, bias.reshape(1, 128), gamma.reshape(1, 128),
      beta.reshape(1, 128), fcW, fcb.reshape(1, 1))


_K = 80


def _sc_edge(src, dst, s16, d16, h, num_heads):
    N = h.shape[0]
    E = src.shape[0]
    NT = 32
    PT = E // NT
    NB = PT // _K
    RPT = N // 16
    mesh = plsc.VectorSubcoreMesh(core_axis_name="c", subcore_axis_name="s")

    @functools.partial(
        pl.kernel,
        mesh=mesh,
        out_type=jax.ShapeDtypeStruct((2, N, 144), jnp.float32),
        scratch_types=[
            pltpu.VMEM((_K,), jnp.int32),
            pltpu.VMEM((_K,), jnp.int32),
            pltpu.VMEM((_K, 16), jnp.float32),
            pltpu.VMEM((_K, 16), jnp.float32),
            pltpu.VMEM((_K, 128), jnp.float32),
            pltpu.VMEM((_K, 144), jnp.float32),
            pltpu.VMEM((16,), jnp.float32),
            pltpu.VMEM_SHARED((N, 144), jnp.float32),
        ],
    )
    def kern(src_hbm, dst_hbm, s_hbm, d_hbm, h_hbm, acc_hbm,
             src_v, dst_v, sbuf, dbuf, rows, u, exscr, A):
        cid = lax.axis_index("c")
        sid = lax.axis_index("s")
        wid = cid * 16 + sid
        base = wid * PT
        zeros16 = jnp.zeros((16,), jnp.float32)
        iota16 = lax.iota(jnp.int32, 16)

        @pl.loop(0, _K)
        def _(k):
            @pl.loop(0, 9)
            def _(j):
                u[k, pl.ds(j * 16, 16)] = zeros16

        r0 = sid * RPT

        @pl.loop(0, RPT // _K)
        def _(b):
            pltpu.sync_copy(u, A.at[pl.ds(r0 + b * _K, _K), :])

        rem = RPT - (RPT // _K) * _K
        if rem:
            pltpu.sync_copy(u.at[pl.ds(0, rem), :],
                            A.at[pl.ds(r0 + RPT - rem, rem), :])
        plsc.subcore_barrier()

        @pl.loop(0, NB)
        def _(b):
            off = base + b * _K
            pltpu.sync_copy(src_hbm.at[pl.ds(off, _K)], src_v)
            pltpu.sync_copy(dst_hbm.at[pl.ds(off, _K)], dst_v)
            pltpu.sync_copy(s_hbm.at[src_v], sbuf)
            pltpu.sync_copy(d_hbm.at[dst_v], dbuf)
            pltpu.sync_copy(h_hbm.at[src_v], rows)

            @pl.loop(0, _K)
            def _(k):
                aq = sbuf[k, :] + dbuf[k, :]
                aq = jnp.maximum(aq, aq * 0.2)
                exq = jnp.exp(aq)
                u[k, pl.ds(128, 16)] = jnp.where(iota16 < num_heads, exq, 0.0)
                exscr[...] = exq
                for c in range(8):
                    hh = (c // 2) if num_heads == 4 else 0
                    w = jnp.full((16,), exscr[hh], jnp.float32)
                    u[k, pl.ds(c * 16, 16)] = rows[k, pl.ds(c * 16, 16)] * w

            pltpu.sync_copy(u, A.at[dst_v], add=True)

        plsc.subcore_barrier()
        pltpu.sync_copy(A.at[pl.ds(r0, RPT), :],
                        acc_hbm.at[cid, pl.ds(r0, RPT), :])

    return kern(src, dst, s16, d16, h)


def _expand_att(att):
    H, C = att.shape
    out = jnp.zeros((128, 16), jnp.float32)
    idx = jnp.arange(H * C)
    return out.at[idx, idx // C].set(att.reshape(-1))


def _head_repeat_mat(H, C):
    out = jnp.zeros((16, 128), jnp.float32)
    idx = jnp.arange(H * C)
    return out.at[idx // C, idx].set(1.0)


def kernel(x, edge_index, W1, att_src1, att_dst1, bias1, gamma1, beta1,
           W2, att_src2, att_dst2, bias2, gamma2, beta2, fcW, fcb):
    src = edge_index[0]
    dst = edge_index[1]

    Asrc1 = _expand_att(att_src1)
    Adst1 = _expand_att(att_dst1)
    R1 = _head_repeat_mat(4, 32)
    Asrc2 = _expand_att(att_src2)
    Adst2 = _expand_att(att_dst2)
    R2 = _head_repeat_mat(1, 128)

    h1, s1, d1 = _tc_pre(x, W1, Asrc1, Adst1)
    acc1 = _sc_edge(src, dst, s1, d1, h1, 4)
    h2, s2, d2 = _tc_mid(acc1, R1, bias1, gamma1, beta1, W2, Asrc2, Adst2)
    acc2 = _sc_edge(src, dst, s2, d2, h2, 1)
    return _tc_post(acc2, R2, bias2, gamma2, beta2, fcW, fcb)


# speedup vs baseline: 17.2159x; 17.2159x over previous
import functools

import jax
import jax.numpy as jnp
from jax import lax
from jax.experimental import pallas as pl
from jax.experimental.pallas import tpu as pltpu
from jax.experimental.pallas import tpu_sc as plsc


def _tc_pre_body(x_ref, w_ref, asrc_ref, adst_ref, h_ref, s_ref, d_ref):
    h = jnp.dot(x_ref[...], w_ref[...], preferred_element_type=jnp.float32)
    h_ref[...] = h
    s_ref[...] = jnp.dot(h, asrc_ref[...], preferred_element_type=jnp.float32)
    d_ref[...] = jnp.dot(h, adst_ref[...], preferred_element_type=jnp.float32)


def _tc_pre(x, W, Asrc, Adst):
    N, F = x.shape
    BLK = 1000
    return pl.pallas_call(
        _tc_pre_body,
        grid=(N // BLK,),
        in_specs=[
            pl.BlockSpec((BLK, F), lambda i: (i, 0)),
            pl.BlockSpec((F, 128), lambda i: (0, 0)),
            pl.BlockSpec((128, 16), lambda i: (0, 0)),
            pl.BlockSpec((128, 16), lambda i: (0, 0)),
        ],
        out_specs=[
            pl.BlockSpec((BLK, 128), lambda i: (i, 0)),
            pl.BlockSpec((BLK, 16), lambda i: (i, 0)),
            pl.BlockSpec((BLK, 16), lambda i: (i, 0)),
        ],
        out_shape=[
            jax.ShapeDtypeStruct((N, 128), jnp.float32),
            jax.ShapeDtypeStruct((N, 16), jnp.float32),
            jax.ShapeDtypeStruct((N, 16), jnp.float32),
        ],
    )(x, W, Asrc, Adst)


_BLK = 1000


def _tc_norm_y_body(acc_ref, r_ref, b_ref, y_ref, sum_ref):
    i = pl.program_id(0)
    P = acc_ref[0] + acc_ref[1]
    hagg = P[:, :128]
    denb = jnp.dot(P[:, 128:144], r_ref[...], preferred_element_type=jnp.float32)
    y = jnp.where(denb > 0.0, hagg / denb, 0.0) + b_ref[...]
    y_ref[...] = y
    part = jnp.sum(y, axis=0, keepdims=True)

    @pl.when(i == 0)
    def _():
        sum_ref[...] = part

    @pl.when(i != 0)
    def _():
        sum_ref[...] += part


def _tc_norm_y(acc, R, bias):
    N = acc.shape[1]
    return pl.pallas_call(
        _tc_norm_y_body,
        grid=(N // _BLK,),
        in_specs=[
            pl.BlockSpec((2, _BLK, 144), lambda i: (0, i, 0)),
            pl.BlockSpec((16, 128), lambda i: (0, 0)),
            pl.BlockSpec((1, 128), lambda i: (0, 0)),
        ],
        out_specs=[
            pl.BlockSpec((_BLK, 128), lambda i: (i, 0)),
            pl.BlockSpec((1, 128), lambda i: (0, 0)),
        ],
        out_shape=[
            jax.ShapeDtypeStruct((N, 128), jnp.float32),
            jax.ShapeDtypeStruct((1, 128), jnp.float32),
        ],
    )(acc, R, bias.reshape(1, 128))


def _tc_var_body(y_ref, sum_ref, v_ref, *, n):
    i = pl.program_id(0)
    mu = sum_ref[...] * (1.0 / n)
    d = y_ref[...] - mu
    part = jnp.sum(d * d, axis=0, keepdims=True)

    @pl.when(i == 0)
    def _():
        v_ref[...] = part

    @pl.when(i != 0)
    def _():
        v_ref[...] += part


def _tc_var(y, ysum):
    N = y.shape[0]
    return pl.pallas_call(
        functools.partial(_tc_var_body, n=N),
        grid=(N // _BLK,),
        in_specs=[
            pl.BlockSpec((_BLK, 128), lambda i: (i, 0)),
            pl.BlockSpec((1, 128), lambda i: (0, 0)),
        ],
        out_specs=pl.BlockSpec((1, 128), lambda i: (0, 0)),
        out_shape=jax.ShapeDtypeStruct((1, 128), jnp.float32),
    )(y, ysum)


def _tc_bn_mat_body(y_ref, sum_ref, v_ref, g_ref, be_ref, w_ref, asrc_ref,
                    adst_ref, h2_ref, s_ref, d_ref, *, n):
    mu = sum_ref[...] * (1.0 / n)
    var = v_ref[...] * (1.0 / n)
    yn = (y_ref[...] - mu) * lax.rsqrt(var + 1e-5) * g_ref[...] + be_ref[...]
    yr = jnp.maximum(yn, 0.0)
    h2 = jnp.dot(yr, w_ref[...], preferred_element_type=jnp.float32)
    h2_ref[...] = h2
    s_ref[...] = jnp.dot(h2, asrc_ref[...], preferred_element_type=jnp.float32)
    d_ref[...] = jnp.dot(h2, adst_ref[...], preferred_element_type=jnp.float32)


def _tc_bn_mat(y, ysum, vsum, gamma, beta, W, Asrc, Adst):
    N = y.shape[0]
    return pl.pallas_call(
        functools.partial(_tc_bn_mat_body, n=N),
        grid=(N // _BLK,),
        in_specs=[
            pl.BlockSpec((_BLK, 128), lambda i: (i, 0)),
            pl.BlockSpec((1, 128), lambda i: (0, 0)),
            pl.BlockSpec((1, 128), lambda i: (0, 0)),
            pl.BlockSpec((1, 128), lambda i: (0, 0)),
            pl.BlockSpec((1, 128), lambda i: (0, 0)),
            pl.BlockSpec((128, 128), lambda i: (0, 0)),
            pl.BlockSpec((128, 16), lambda i: (0, 0)),
            pl.BlockSpec((128, 16), lambda i: (0, 0)),
        ],
        out_specs=[
            pl.BlockSpec((_BLK, 128), lambda i: (i, 0)),
            pl.BlockSpec((_BLK, 16), lambda i: (i, 0)),
            pl.BlockSpec((_BLK, 16), lambda i: (i, 0)),
        ],
        out_shape=[
            jax.ShapeDtypeStruct((N, 128), jnp.float32),
            jax.ShapeDtypeStruct((N, 16), jnp.float32),
            jax.ShapeDtypeStruct((N, 16), jnp.float32),
        ],
    )(y, ysum, vsum, gamma.reshape(1, 128), beta.reshape(1, 128), W, Asrc,
      Adst)


def _tc_mid(acc, R, bias, gamma, beta, W, Asrc, Adst):
    y, ysum = _tc_norm_y(acc, R, bias)
    vsum = _tc_var(y, ysum)
    return _tc_bn_mat(y, ysum, vsum, gamma, beta, W, Asrc, Adst)


def _tc_bn_fc_body(y_ref, sum_ref, v_ref, g_ref, be_ref, fcw_ref, fcb_ref,
                   out_ref, *, n):
    mu = sum_ref[...] * (1.0 / n)
    var = v_ref[...] * (1.0 / n)
    yn = (y_ref[...] - mu) * lax.rsqrt(var + 1e-5) * g_ref[...] + be_ref[...]
    yr = jnp.maximum(yn, 0.0)
    o = jnp.dot(yr, fcw_ref[...], preferred_element_type=jnp.float32)
    out_ref[...] = jax.nn.sigmoid(o + fcb_ref[0, 0])


def _tc_post(acc, R, bias, gamma, beta, fcW, fcb):
    N = acc.shape[1]
    y, ysum = _tc_norm_y(acc, R, bias)
    vsum = _tc_var(y, ysum)
    return pl.pallas_call(
        functools.partial(_tc_bn_fc_body, n=N),
        grid=(N // _BLK,),
        in_specs=[
            pl.BlockSpec((_BLK, 128), lambda i: (i, 0)),
            pl.BlockSpec((1, 128), lambda i: (0, 0)),
            pl.BlockSpec((1, 128), lambda i: (0, 0)),
            pl.BlockSpec((1, 128), lambda i: (0, 0)),
            pl.BlockSpec((1, 128), lambda i: (0, 0)),
            pl.BlockSpec((128, 1), lambda i: (0, 0)),
            pl.BlockSpec((1, 1), lambda i: (0, 0)),
        ],
        out_specs=pl.BlockSpec((_BLK, 1), lambda i: (i, 0)),
        out_shape=jax.ShapeDtypeStruct((N, 1), jnp.float32),
    )(y, ysum, vsum, gamma.reshape(1, 128), beta.reshape(1, 128), fcW,
      fcb.reshape(1, 1))


_K = 80


def _sc_edge(src, dst, s16, d16, h, num_heads):
    N = h.shape[0]
    E = src.shape[0]
    NT = 32
    PT = E // NT
    NB = PT // _K
    RPT = (N // 16) // 8 * 8
    mesh = plsc.VectorSubcoreMesh(core_axis_name="c", subcore_axis_name="s")

    @functools.partial(
        pl.kernel,
        mesh=mesh,
        compiler_params=pltpu.CompilerParams(use_tc_tiling_on_sc=False),
        out_type=jax.ShapeDtypeStruct((2, N, 144), jnp.float32),
        scratch_types=[
            pltpu.VMEM((_K,), jnp.int32),
            pltpu.VMEM((_K,), jnp.int32),
            pltpu.VMEM((_K, 16), jnp.float32),
            pltpu.VMEM((_K, 16), jnp.float32),
            pltpu.VMEM((_K, 128), jnp.float32),
            pltpu.VMEM((_K, 144), jnp.float32),
            pltpu.VMEM_SHARED((N, 144), jnp.float32),
        ],
    )
    def kern(src_hbm, dst_hbm, s_hbm, d_hbm, h_hbm, acc_hbm,
             src_v, dst_v, sbuf, dbuf, rows, u, A):
        cid = lax.axis_index("c")
        sid = lax.axis_index("s")
        wid = cid * 16 + sid
        base = wid * PT
        zeros16 = jnp.zeros((16,), jnp.float32)
        iota16 = lax.iota(jnp.int32, 16)

        @pl.loop(0, _K)
        def _(k):
            @pl.loop(0, 9)
            def _(j):
                u[k, pl.ds(j * 16, 16)] = zeros16

        r0 = sid * RPT

        @pl.loop(0, RPT // _K)
        def _(b):
            pltpu.sync_copy(u, A.at[pl.ds(r0 + b * _K, _K), :])

        rem = RPT - (RPT // _K) * _K
        if rem:
            pltpu.sync_copy(u.at[pl.ds(0, rem), :],
                            A.at[pl.ds(r0 + RPT - rem, rem), :])

        nlast = N - 16 * RPT
        if nlast:
            @pl.when(sid == 15)
            def _():
                pltpu.sync_copy(u.at[pl.ds(0, nlast), :],
                                A.at[pl.ds(16 * RPT, nlast), :])
        plsc.subcore_barrier()

        @pl.loop(0, NB)
        def _(b):
            off = base + b * _K
            pltpu.sync_copy(src_hbm.at[pl.ds(off, _K)], src_v)
            pltpu.sync_copy(dst_hbm.at[pl.ds(off, _K)], dst_v)
            pltpu.sync_copy(s_hbm.at[src_v], sbuf)
            pltpu.sync_copy(d_hbm.at[dst_v], dbuf)
            pltpu.sync_copy(h_hbm.at[src_v], rows)

            @pl.loop(0, _K)
            def _(k):
                aq = sbuf[k, :] + dbuf[k, :]
                aq = jnp.maximum(aq, aq * 0.2)
                exq = jnp.exp(aq)
                u[k, pl.ds(128, 16)] = jnp.where(iota16 < num_heads, exq, 0.0)
                for c in range(8):
                    hh = (c // 2) if num_heads == 4 else 0
                    w = jnp.full((16,), exq[hh], jnp.float32)
                    u[k, pl.ds(c * 16, 16)] = rows[k, pl.ds(c * 16, 16)] * w

            pltpu.sync_copy(u, A.at[dst_v], add=True)

        plsc.subcore_barrier()

        @pl.loop(0, RPT // _K)
        def _(b):
            pltpu.sync_copy(A.at[pl.ds(r0 + b * _K, _K), :], u)
            pltpu.sync_copy(u, acc_hbm.at[cid, pl.ds(r0 + b * _K, _K), :])

        if rem:
            pltpu.sync_copy(A.at[pl.ds(r0 + RPT - rem, rem), :],
                            u.at[pl.ds(0, rem), :])
            pltpu.sync_copy(u.at[pl.ds(0, rem), :],
                            acc_hbm.at[cid, pl.ds(r0 + RPT - rem, rem), :])
        if nlast:
            @pl.when(sid == 15)
            def _():
                pltpu.sync_copy(A.at[pl.ds(16 * RPT, nlast), :],
                                u.at[pl.ds(0, nlast), :])
                pltpu.sync_copy(u.at[pl.ds(0, nlast), :],
                                acc_hbm.at[cid, pl.ds(16 * RPT, nlast), :])

    return kern(src, dst, s16, d16, h)


def _expand_att(att):
    H, C = att.shape
    rows = jnp.arange(H * C)[:, None]
    cols = jnp.arange(16)[None, :]
    return jnp.where(cols == rows // C, att.reshape(-1)[:, None], 0.0)


def _head_repeat_mat(H, C):
    rows = jnp.arange(16)[:, None]
    cols = jnp.arange(H * C)[None, :]
    return jnp.where(rows == cols // C, 1.0, 0.0).astype(jnp.float32)


def kernel(x, edge_index, W1, att_src1, att_dst1, bias1, gamma1, beta1,
           W2, att_src2, att_dst2, bias2, gamma2, beta2, fcW, fcb):
    src = edge_index[0]
    dst = edge_index[1]

    Asrc1 = _expand_att(att_src1)
    Adst1 = _expand_att(att_dst1)
    R1 = _head_repeat_mat(4, 32)
    Asrc2 = _expand_att(att_src2)
    Adst2 = _expand_att(att_dst2)
    R2 = _head_repeat_mat(1, 128)

    h1, s1, d1 = _tc_pre(x, W1, Asrc1, Adst1)
    acc1 = _sc_edge(src, dst, s1, d1, h1, 4)
    h2, s2, d2 = _tc_mid(acc1, R1, bias1, gamma1, beta1, W2, Asrc2, Adst2)
    acc2 = _sc_edge(src, dst, s2, d2, h2, 1)
    return _tc_post(acc2, R2, bias2, gamma2, beta2, fcW, fcb)
